# baseline (speedup 1.0000x reference)
import jax
import jax.numpy as jnp
from jax import lax
from jax.experimental import pallas as pl
from jax.experimental.pallas import tpu as pltpu
from jax.experimental.pallas import tpu_sc as plsc

DEPTH = 1000
N_ROWS = 4096
N_COLS = 20
TOT = N_ROWS * N_COLS

NUM_CORES = 2
NUM_SUBCORES = 16
NW = NUM_CORES * NUM_SUBCORES
QPW = TOT // NW

FILL_ROWS = 128


def _fill_body(out_ref):
    out_ref[...] = jnp.zeros_like(out_ref)


def _zero_filled():
    return pl.pallas_call(
        _fill_body,
        grid=(N_ROWS // FILL_ROWS,),
        out_specs=pl.BlockSpec((FILL_ROWS, N_COLS, DEPTH), lambda i: (i, 0, 0)),
        out_shape=jax.ShapeDtypeStruct((N_ROWS, N_COLS, DEPTH), jnp.float32),
    )()


def _scatter_stateful(refs):
    idx_ref, pat_ref, out_ref = refs
    mesh = plsc.VectorSubcoreMesh(core_axis_name="c", subcore_axis_name="s")

    @pl.core_map(
        mesh,
        compiler_params=pltpu.CompilerParams(
            use_tc_tiling_on_sc=False, needs_layout_passes=False
        ),
        scratch_shapes=[
            pltpu.VMEM((QPW,), jnp.int32),
            pltpu.VMEM((8, 1, 1, 8), jnp.float32),
            pltpu.SemaphoreType.DMA,
        ],
    )
    def _(idx_v, pat_v, sem):
        c = lax.axis_index("c")
        s = lax.axis_index("s")
        wid = s * NUM_CORES + c
        base_q = wid * QPW
        pltpu.sync_copy(idx_ref.at[pl.ds(base_q, QPW)], idx_v)
        pltpu.sync_copy(pat_ref, pat_v)

        def one_body(j, carry):
            vec = idx_v[pl.ds(j * 16, 16)]
            for l in range(16):
                v = vec[l]
                q = base_q + j * 16 + l
                n = q // N_COLS
                m = lax.rem(q, N_COLS)
                d8 = (v // 8) * 8
                pltpu.make_async_copy(
                    pat_v.at[lax.rem(v, 8)],
                    out_ref.at[pl.ds(n, 1), pl.ds(m, 1), pl.ds(d8, 8)],
                    sem,
                ).start()
            return carry

        lax.fori_loop(0, QPW // 16, one_body, 0)

        def drain_body(j, carry):
            pltpu.make_async_copy(
                pat_v.at[0],
                out_ref.at[pl.ds(base_q // N_COLS, 1), pl.ds(0, 1), pl.ds(0, 8)],
                sem,
            ).wait()
            return carry

        lax.fori_loop(0, QPW, drain_body, 0)


def kernel(inputs):
    idx = inputs.astype(jnp.int32).reshape(TOT)
    patterns = jnp.eye(8, dtype=jnp.float32).reshape(8, 1, 1, 8)
    init = _zero_filled()
    _, _, out = pl.run_state(_scatter_stateful)((idx, patterns, init))
    return out


# speedup vs baseline: 2.4400x; 2.4400x over previous
import jax
import jax.numpy as jnp
from jax import lax
from jax.experimental import pallas as pl
from jax.experimental.pallas import tpu as pltpu

DEPTH = 1000
ROWS_PER_BLOCK = 128


def _onehot_block(inp_ref, out_ref):
    idx = inp_ref[...]
    iota = lax.broadcasted_iota(jnp.int32, (idx.shape[0], idx.shape[1], DEPTH), 2)
    out_ref[...] = (idx[:, :, None] == iota).astype(jnp.float32)


def kernel(inputs):
    n, m = inputs.shape
    r = ROWS_PER_BLOCK
    return pl.pallas_call(
        _onehot_block,
        grid=(n // r,),
        in_specs=[pl.BlockSpec((r, m), lambda i: (i, 0))],
        out_specs=pl.BlockSpec((r, m, DEPTH), lambda i: (i, 0, 0)),
        out_shape=jax.ShapeDtypeStruct((n, m, DEPTH), jnp.float32),
        compiler_params=pltpu.CompilerParams(
            dimension_semantics=("parallel",),
        ),
    )(inputs.astype(jnp.int32))
